# Initial kernel scaffold; baseline (speedup 1.0000x reference)
import jax
import jax.numpy as jnp
from jax.experimental import pallas as pl


def kernel(x, edge_index, W0, b0, W1, b1, W2, b2):
    raise NotImplementedError("write your pallas kernel here")


# speedup vs baseline: 3.0585x; 3.0585x over previous
import functools

import jax
import jax.numpy as jnp
from jax.experimental import pallas as pl
from jax.experimental.pallas import tpu as pltpu

N = 10000
E = 320000
D = 128
C = 40


def _mm_body(x_ref, w_ref, s_ref, o_ref):
    o_ref[...] = s_ref[...] * jnp.dot(
        x_ref[...],
        w_ref[...],
        preferred_element_type=jnp.float32,
        precision=jax.lax.Precision.HIGHEST,
    )


def _scaled_matmul(x, w, s):
    return pl.pallas_call(
        _mm_body,
        out_shape=jax.ShapeDtypeStruct((x.shape[0], w.shape[1]), jnp.float32),
    )(x, w, s)


def _fin_body(x_ref, w_ref, b_ref, o_ref):
    o_ref[...] = (
        jnp.dot(
            x_ref[...],
            w_ref[...],
            preferred_element_type=jnp.float32,
            precision=jax.lax.Precision.HIGHEST,
        )
        + b_ref[...]
    )


def _final_matmul(x, w, b):
    return pl.pallas_call(
        _fin_body,
        out_shape=jax.ShapeDtypeStruct((x.shape[0], w.shape[1]), jnp.float32),
    )(x, w, b[None, :])


def kernel(x, edge_index, W0, b0, W1, b1, W2, b2):
    src = edge_index[0]
    dst = edge_index[1]

    deg = jnp.zeros((N,), jnp.float32).at[dst].add(1.0)
    dis_nl = jnp.where(deg > 0, jax.lax.rsqrt(jnp.maximum(deg, 1e-12)), 0.0)
    dis_wl = jax.lax.rsqrt(deg + 1.0)

    def agg(p):
        return jax.ops.segment_sum(p[src], dst, num_segments=N)

    p0 = _scaled_matmul(x, W0, dis_nl[:, None])
    x1 = jax.nn.relu(dis_nl[:, None] * agg(p0) + b0)
    p1 = _scaled_matmul(x1, W1, dis_nl[:, None])
    x2 = jax.nn.relu(dis_nl[:, None] * agg(p1) + b1)
    p2 = dis_wl[:, None] * x2
    q = dis_wl[:, None] * (agg(p2) + p2)
    return _final_matmul(q, W2, b2)


# speedup vs baseline: 18.7609x; 6.1340x over previous
import functools

import jax
import jax.numpy as jnp
from jax import lax
from jax.experimental import pallas as pl
from jax.experimental.pallas import tpu as pltpu
from jax.experimental.pallas import tpu_sc as plsc

N = 10000
E = 320000
D = 128
NCLS = 40

_TILES = 32
_NS = 16
_EPT = E // _TILES
_CH = 80
_NCHUNK = _EPT // _CH
_RPT = N // _NS

_HI = jax.lax.Precision.HIGHEST


@functools.cache
def _sc_agg():
    mesh = plsc.VectorSubcoreMesh(core_axis_name="c", subcore_axis_name="s")

    @functools.partial(
        pl.kernel,
        out_type=jax.ShapeDtypeStruct((2, N, D), jnp.float32),
        mesh=mesh,
        scratch_types=[
            pltpu.VMEM((_NCHUNK, _CH), jnp.int32),
            pltpu.VMEM((_NCHUNK, _CH), jnp.int32),
            pltpu.VMEM((_CH, D), jnp.float32),
            pltpu.VMEM_SHARED((N, D), jnp.float32),
        ],
    )
    def agg(p_hbm, srcr_hbm, dstr_hbm, zmat_hbm, out_hbm, src_v, dst_v, rows_v, acc):
        c = lax.axis_index("c")
        s = lax.axis_index("s")
        t = c * _NS + s

        @pl.when(s == 0)
        def _():
            pltpu.sync_copy(zmat_hbm, acc)

        plsc.subcore_barrier()
        pltpu.sync_copy(srcr_hbm.at[t], src_v)
        pltpu.sync_copy(dstr_hbm.at[t], dst_v)

        def step(k, carry):
            pltpu.sync_copy(p_hbm.at[src_v.at[k]], rows_v)
            pltpu.sync_copy(rows_v, acc.at[dst_v.at[k]], add=True)
            return carry

        lax.fori_loop(0, _NCHUNK, step, 0)
        plsc.subcore_barrier()

        @pl.when(s == 0)
        def _():
            pltpu.sync_copy(acc, out_hbm.at[c])

    return agg


@functools.cache
def _sc_deg():
    mesh = plsc.VectorSubcoreMesh(core_axis_name="c", subcore_axis_name="s")

    @functools.partial(
        pl.kernel,
        out_type=jax.ShapeDtypeStruct((2, N), jnp.float32),
        mesh=mesh,
        scratch_types=[
            pltpu.VMEM((_NCHUNK, _CH), jnp.int32),
            pltpu.VMEM((_CH,), jnp.float32),
            pltpu.VMEM_SHARED((N,), jnp.float32),
        ],
    )
    def deg(dstr_hbm, zvec_hbm, out_hbm, dst_v, ones_v, acc):
        c = lax.axis_index("c")
        s = lax.axis_index("s")
        t = c * _NS + s

        @pl.when(s == 0)
        def _():
            pltpu.sync_copy(zvec_hbm, acc)

        for i in range(_CH // 16):
            ones_v[pl.ds(i * 16, 16)] = jnp.full((16,), 1.0, jnp.float32)

        plsc.subcore_barrier()
        pltpu.sync_copy(dstr_hbm.at[t], dst_v)

        def step(k, carry):
            pltpu.sync_copy(ones_v, acc.at[dst_v.at[k]], add=True)
            return carry

        lax.fori_loop(0, _NCHUNK, step, 0)
        plsc.subcore_barrier()

        @pl.when(s == 0)
        def _():
            pltpu.sync_copy(acc, out_hbm.at[c])

    return deg


def _tc1_body(x_ref, w_ref, dega_ref, degb_ref, p_ref, disnl_ref, diswl_ref):
    deg = dega_ref[...] + degb_ref[...]
    disnl = jnp.where(deg > 0, lax.rsqrt(jnp.maximum(deg, 1e-12)), 0.0)
    diswl = lax.rsqrt(deg + 1.0)
    disnl_ref[...] = disnl
    diswl_ref[...] = diswl
    p_ref[...] = disnl * jnp.dot(
        x_ref[...], w_ref[...], preferred_element_type=jnp.float32, precision=_HI
    )


def _tc2_body(a0_ref, a1_ref, disnl_ref, b_ref, w_ref, p_ref):
    disnl = disnl_ref[...]
    x1 = jnp.maximum(disnl * (a0_ref[...] + a1_ref[...]) + b_ref[...], 0.0)
    p_ref[...] = disnl * jnp.dot(
        x1, w_ref[...], preferred_element_type=jnp.float32, precision=_HI
    )


def _tc3_body(a0_ref, a1_ref, disnl_ref, diswl_ref, b_ref, p_ref):
    x2 = jnp.maximum(
        disnl_ref[...] * (a0_ref[...] + a1_ref[...]) + b_ref[...], 0.0
    )
    p_ref[...] = diswl_ref[...] * x2


def _tc4_body(a0_ref, a1_ref, p2_ref, diswl_ref, w_ref, b_ref, o_ref):
    q = diswl_ref[...] * (a0_ref[...] + a1_ref[...] + p2_ref[...])
    o_ref[...] = (
        jnp.dot(q, w_ref[...], preferred_element_type=jnp.float32, precision=_HI)
        + b_ref[...]
    )


def _call(body, n_out, out_shapes):
    return pl.pallas_call(
        body,
        out_shape=[jax.ShapeDtypeStruct(s, jnp.float32) for s in out_shapes]
        if n_out > 1
        else jax.ShapeDtypeStruct(out_shapes[0], jnp.float32),
    )


def kernel(x, edge_index, W0, b0, W1, b1, W2, b2):
    srcr = edge_index[0].reshape(_TILES, _NCHUNK, _CH)
    dstr = edge_index[1].reshape(_TILES, _NCHUNK, _CH)
    zvec = jnp.zeros((N,), jnp.float32)
    zmat = jnp.zeros((N, D), jnp.float32)

    degp = _sc_deg()(dstr, zvec)
    dega = degp[0][:, None]
    degb = degp[1][:, None]

    p0, disnl, diswl = _call(_tc1_body, 3, [(N, D), (N, 1), (N, 1)])(
        x, W0, dega, degb
    )

    a1 = _sc_agg()(p0, srcr, dstr, zmat)
    p1 = _call(_tc2_body, 1, [(N, D)])(a1[0], a1[1], disnl, b0[None, :], W1)

    a2 = _sc_agg()(p1, srcr, dstr, zmat)
    p2 = _call(_tc3_body, 1, [(N, D)])(a2[0], a2[1], disnl, diswl, b1[None, :])

    a3 = _sc_agg()(p2, srcr, dstr, zmat)
    out = _call(_tc4_body, 1, [(N, NCLS)])(
        a3[0], a3[1], p2, diswl, W2, b2[None, :]
    )
    return out


# speedup vs baseline: 26.9090x; 1.4343x over previous
import functools

import jax
import jax.numpy as jnp
from jax import lax
from jax.experimental import pallas as pl
from jax.experimental.pallas import tpu as pltpu
from jax.experimental.pallas import tpu_sc as plsc

N = 10000
E = 320000
D = 128
NCLS = 40

_TILES = 32
_NS = 16
_EPT = E // _TILES
_CH = 80
_NCHUNK = _EPT // _CH

_ACH = 125
_ANCH = _EPT // _ACH
_W = 8
_NG = _ANCH // _W

_HI = jax.lax.Precision.HIGHEST


@functools.cache
def _sc_agg():
    mesh = plsc.VectorSubcoreMesh(core_axis_name="c", subcore_axis_name="s")

    @functools.partial(
        pl.kernel,
        out_type=jax.ShapeDtypeStruct((2, N, D), jnp.float32),
        mesh=mesh,
        scratch_types=[
            pltpu.VMEM((_W, _ACH), jnp.int32),
            pltpu.VMEM((_W, _ACH), jnp.int32),
            pltpu.VMEM((_W, _ACH), jnp.int32),
            pltpu.VMEM((_W, _ACH), jnp.int32),
            pltpu.VMEM((_ACH, D), jnp.float32),
            pltpu.VMEM((_ACH, D), jnp.float32),
            pltpu.VMEM_SHARED((N, D), jnp.float32),
            pltpu.SemaphoreType.DMA((2,)),
            pltpu.SemaphoreType.DMA((2,)),
            pltpu.SemaphoreType.DMA((2,)),
        ],
    )
    def agg(p_hbm, srcr_hbm, dstr_hbm, zmat_hbm, out_hbm,
            si0, si1, di0, di1, rb0, rb1, acc, gsem, ssem, isem):
        sibs = [si0, si1]
        dibs = [di0, di1]
        rbs = [rb0, rb1]
        c = lax.axis_index("c")
        s = lax.axis_index("s")
        t = c * _NS + s

        @pl.when(s == 0)
        def _():
            pltpu.sync_copy(zmat_hbm, acc)

        plsc.subcore_barrier()

        def g_start(k):
            g, j, b = k // _W, k % _W, k % 2
            pltpu.async_copy(p_hbm.at[sibs[g % 2].at[j]], rbs[b], gsem.at[b])

        def g_wait(k):
            g, j, b = k // _W, k % _W, k % 2
            pltpu.make_async_copy(p_hbm.at[sibs[g % 2].at[j]], rbs[b],
                                  gsem.at[b]).wait()

        def s_start(k):
            g, j, b = k // _W, k % _W, k % 2
            pltpu.async_copy(rbs[b], acc.at[dibs[g % 2].at[j]], ssem.at[b],
                             add=True)

        def s_wait(k):
            g, j, b = k // _W, k % _W, k % 2
            pltpu.make_async_copy(rbs[b], acc.at[dibs[g % 2].at[j]],
                                  ssem.at[b]).wait()

        def i_start(g):
            ib = g % 2
            pltpu.async_copy(srcr_hbm.at[t, g], sibs[ib], isem.at[ib])
            pltpu.async_copy(dstr_hbm.at[t, g], dibs[ib], isem.at[ib])

        def i_wait(g):
            ib = g % 2
            pltpu.make_async_copy(srcr_hbm.at[t, g], sibs[ib],
                                  isem.at[ib]).wait()
            pltpu.make_async_copy(dstr_hbm.at[t, g], dibs[ib],
                                  isem.at[ib]).wait()

        pltpu.sync_copy(srcr_hbm.at[t, 0], si0)
        pltpu.sync_copy(dstr_hbm.at[t, 0], di0)
        i_start(1)
        g_start(0)

        for k in range(_ANCH):
            g_wait(k)
            s_start(k)
            if k > 0:
                s_wait(k - 1)
            if k % _W == 0 and k > 0:
                if k // _W + 1 < _NG:
                    i_start(k // _W + 1)
            if k + 1 < _ANCH:
                if (k + 1) % _W == 0:
                    i_wait(k // _W + 1)
                g_start(k + 1)

        s_wait(_ANCH - 1)
        plsc.subcore_barrier()

        @pl.when(s == 0)
        def _():
            pltpu.sync_copy(acc, out_hbm.at[c])

    return agg


@functools.cache
def _sc_deg():
    mesh = plsc.VectorSubcoreMesh(core_axis_name="c", subcore_axis_name="s")

    @functools.partial(
        pl.kernel,
        out_type=jax.ShapeDtypeStruct((2, N), jnp.float32),
        mesh=mesh,
        scratch_types=[
            pltpu.VMEM((_NCHUNK, _CH), jnp.int32),
            pltpu.VMEM((_CH,), jnp.float32),
            pltpu.VMEM_SHARED((N,), jnp.float32),
        ],
    )
    def deg(dstr_hbm, zvec_hbm, out_hbm, dst_v, ones_v, acc):
        c = lax.axis_index("c")
        s = lax.axis_index("s")
        t = c * _NS + s

        @pl.when(s == 0)
        def _():
            pltpu.sync_copy(zvec_hbm, acc)

        for i in range(_CH // 16):
            ones_v[pl.ds(i * 16, 16)] = jnp.full((16,), 1.0, jnp.float32)

        plsc.subcore_barrier()
        pltpu.sync_copy(dstr_hbm.at[t], dst_v)

        def step(k, carry):
            pltpu.sync_copy(ones_v, acc.at[dst_v.at[k]], add=True)
            return carry

        lax.fori_loop(0, _NCHUNK, step, 0)
        plsc.subcore_barrier()

        @pl.when(s == 0)
        def _():
            pltpu.sync_copy(acc, out_hbm.at[c])

    return deg


def _tc1_body(x_ref, w_ref, dega_ref, degb_ref, p_ref, disnl_ref, diswl_ref):
    deg = dega_ref[...] + degb_ref[...]
    disnl = jnp.where(deg > 0, lax.rsqrt(jnp.maximum(deg, 1e-12)), 0.0)
    diswl = lax.rsqrt(deg + 1.0)
    disnl_ref[...] = disnl
    diswl_ref[...] = diswl
    p_ref[...] = disnl * jnp.dot(
        x_ref[...], w_ref[...], preferred_element_type=jnp.float32, precision=_HI
    )


def _tc2_body(a0_ref, a1_ref, disnl_ref, b_ref, w_ref, p_ref):
    disnl = disnl_ref[...]
    x1 = jnp.maximum(disnl * (a0_ref[...] + a1_ref[...]) + b_ref[...], 0.0)
    p_ref[...] = disnl * jnp.dot(
        x1, w_ref[...], preferred_element_type=jnp.float32, precision=_HI
    )


def _tc3_body(a0_ref, a1_ref, disnl_ref, diswl_ref, b_ref, p_ref):
    x2 = jnp.maximum(
        disnl_ref[...] * (a0_ref[...] + a1_ref[...]) + b_ref[...], 0.0
    )
    p_ref[...] = diswl_ref[...] * x2


def _tc4_body(a0_ref, a1_ref, p2_ref, diswl_ref, w_ref, b_ref, o_ref):
    q = diswl_ref[...] * (a0_ref[...] + a1_ref[...] + p2_ref[...])
    o_ref[...] = (
        jnp.dot(q, w_ref[...], preferred_element_type=jnp.float32, precision=_HI)
        + b_ref[...]
    )


def _call(body, n_out, out_shapes):
    return pl.pallas_call(
        body,
        out_shape=[jax.ShapeDtypeStruct(s, jnp.float32) for s in out_shapes]
        if n_out > 1
        else jax.ShapeDtypeStruct(out_shapes[0], jnp.float32),
    )


def kernel(x, edge_index, W0, b0, W1, b1, W2, b2):
    srcr = edge_index[0].reshape(_TILES, _NG, _W, _ACH)
    dstr = edge_index[1].reshape(_TILES, _NG, _W, _ACH)
    dstr_deg = edge_index[1].reshape(_TILES, _NCHUNK, _CH)
    zvec = jnp.zeros((N,), jnp.float32)
    zmat = jnp.zeros((N, D), jnp.float32)

    degp = _sc_deg()(dstr_deg, zvec)
    dega = degp[0][:, None]
    degb = degp[1][:, None]

    p0, disnl, diswl = _call(_tc1_body, 3, [(N, D), (N, 1), (N, 1)])(
        x, W0, dega, degb
    )

    a1 = _sc_agg()(p0, srcr, dstr, zmat)
    p1 = _call(_tc2_body, 1, [(N, D)])(a1[0], a1[1], disnl, b0[None, :], W1)

    a2 = _sc_agg()(p1, srcr, dstr, zmat)
    p2 = _call(_tc3_body, 1, [(N, D)])(a2[0], a2[1], disnl, diswl, b1[None, :])

    a3 = _sc_agg()(p2, srcr, dstr, zmat)
    out = _call(_tc4_body, 1, [(N, NCLS)])(
        a3[0], a3[1], p2, diswl, W2, b2[None, :]
    )
    return out


# speedup vs baseline: 31.7764x; 1.1809x over previous
import functools

import jax
import jax.numpy as jnp
from jax import lax
from jax.experimental import pallas as pl
from jax.experimental.pallas import tpu as pltpu
from jax.experimental.pallas import tpu_sc as plsc

N = 10000
E = 320000
D = 128
NCLS = 40

_TILES = 32
_NS = 16
_EPT = E // _TILES
_CH = 80
_NCHUNK = _EPT // _CH

_ACH = 80
_ANCH = _EPT // _ACH
_W = 5
_NG = _ANCH // _W
_NB = 4

_HI = jax.lax.Precision.HIGHEST


@functools.cache
def _sc_agg():
    mesh = plsc.VectorSubcoreMesh(core_axis_name="c", subcore_axis_name="s")

    @functools.partial(
        pl.kernel,
        out_type=jax.ShapeDtypeStruct((2, N, D), jnp.float32),
        mesh=mesh,
        scratch_types=[
            pltpu.VMEM((_W, _ACH), jnp.int32),
            pltpu.VMEM((_W, _ACH), jnp.int32),
            pltpu.VMEM((_W, _ACH), jnp.int32),
            pltpu.VMEM((_W, _ACH), jnp.int32),
            pltpu.VMEM((_ACH, D), jnp.float32),
            pltpu.VMEM((_ACH, D), jnp.float32),
            pltpu.VMEM((_ACH, D), jnp.float32),
            pltpu.VMEM((_ACH, D), jnp.float32),
            pltpu.VMEM_SHARED((N, D), jnp.float32),
            pltpu.SemaphoreType.DMA((_NB,)),
            pltpu.SemaphoreType.DMA((_NB,)),
            pltpu.SemaphoreType.DMA((2,)),
        ],
    )
    def agg(p_hbm, srcr_hbm, dstr_hbm, zmat_hbm, out_hbm,
            si0, si1, di0, di1, rb0, rb1, rb2, rb3, acc, gsem, ssem, isem):
        sibs = [si0, si1]
        dibs = [di0, di1]
        rbs = [rb0, rb1, rb2, rb3]
        c = lax.axis_index("c")
        s = lax.axis_index("s")
        t = c * _NS + s

        @pl.when(s == 0)
        def _():
            pltpu.sync_copy(zmat_hbm, acc)

        plsc.subcore_barrier()

        def g_start(k):
            g, j, b = k // _W, k % _W, k % _NB
            pltpu.async_copy(p_hbm.at[sibs[g % 2].at[j]], rbs[b], gsem.at[b])

        def g_wait(k):
            g, j, b = k // _W, k % _W, k % _NB
            pltpu.make_async_copy(p_hbm.at[sibs[g % 2].at[j]], rbs[b],
                                  gsem.at[b]).wait()

        def s_start(k):
            g, j, b = k // _W, k % _W, k % _NB
            pltpu.async_copy(rbs[b], acc.at[dibs[g % 2].at[j]], ssem.at[b],
                             add=True)

        def s_wait(k):
            g, j, b = k // _W, k % _W, k % _NB
            pltpu.make_async_copy(rbs[b], acc.at[dibs[g % 2].at[j]],
                                  ssem.at[b]).wait()

        def i_start(g):
            ib = g % 2
            pltpu.async_copy(srcr_hbm.at[t, g], sibs[ib], isem.at[ib])
            pltpu.async_copy(dstr_hbm.at[t, g], dibs[ib], isem.at[ib])

        def i_wait(g):
            ib = g % 2
            pltpu.make_async_copy(srcr_hbm.at[t, g], sibs[ib],
                                  isem.at[ib]).wait()
            pltpu.make_async_copy(dstr_hbm.at[t, g], dibs[ib],
                                  isem.at[ib]).wait()

        pltpu.sync_copy(srcr_hbm.at[t, 0], si0)
        pltpu.sync_copy(dstr_hbm.at[t, 0], di0)
        i_start(1)
        for k in range(_NB - 1):
            g_start(k)

        for k in range(_ANCH):
            g_wait(k)
            s_start(k)
            if k > 0:
                s_wait(k - 1)
            if k % _W == 0 and k > 0 and k // _W + 1 < _NG:
                i_start(k // _W + 1)
            kn = k + _NB - 1
            if kn < _ANCH:
                if kn % _W < _NB - 1 and kn // _W > 0:
                    if kn % _W == 0:
                        i_wait(kn // _W)
                g_start(kn)

        s_wait(_ANCH - 1)
        plsc.subcore_barrier()

        @pl.when(s == 0)
        def _():
            pltpu.sync_copy(acc, out_hbm.at[c])

    return agg


@functools.cache
def _sc_deg():
    mesh = plsc.VectorSubcoreMesh(core_axis_name="c", subcore_axis_name="s")

    @functools.partial(
        pl.kernel,
        out_type=jax.ShapeDtypeStruct((2, N), jnp.float32),
        mesh=mesh,
        scratch_types=[
            pltpu.VMEM((_W, _ACH), jnp.int32),
            pltpu.VMEM((_W, _ACH), jnp.int32),
            pltpu.VMEM((_ACH,), jnp.float32),
            pltpu.VMEM_SHARED((N,), jnp.float32),
            pltpu.SemaphoreType.DMA((2,)),
        ],
    )
    def deg(dstr_hbm, zvec_hbm, out_hbm, di0, di1, ones_v, acc, isem):
        dibs = [di0, di1]
        c = lax.axis_index("c")
        s = lax.axis_index("s")
        t = c * _NS + s

        @pl.when(s == 0)
        def _():
            pltpu.sync_copy(zvec_hbm, acc)

        for i in range(_ACH // 16):
            ones_v[pl.ds(i * 16, 16)] = jnp.full((16,), 1.0, jnp.float32)

        plsc.subcore_barrier()
        pltpu.sync_copy(dstr_hbm.at[t, 0], di0)

        def i_start(g):
            pltpu.async_copy(dstr_hbm.at[t, g], dibs[g % 2], isem.at[g % 2])

        def i_wait(g):
            pltpu.make_async_copy(dstr_hbm.at[t, g], dibs[g % 2],
                                  isem.at[g % 2]).wait()

        i_start(1)
        for g in range(_NG):
            if g > 0:
                i_wait(g)
            for j in range(_W):
                pltpu.sync_copy(ones_v, acc.at[dibs[g % 2].at[j]], add=True)
            if g + 2 < _NG:
                i_start(g + 2)

        plsc.subcore_barrier()

        @pl.when(s == 0)
        def _():
            pltpu.sync_copy(acc, out_hbm.at[c])

    return deg


def _tc1_body(x_ref, w_ref, dega_ref, degb_ref, p_ref, disnl_ref, diswl_ref):
    deg = dega_ref[...] + degb_ref[...]
    disnl = jnp.where(deg > 0, lax.rsqrt(jnp.maximum(deg, 1e-12)), 0.0)
    diswl = lax.rsqrt(deg + 1.0)
    disnl_ref[...] = disnl
    diswl_ref[...] = diswl
    p_ref[...] = disnl * jnp.dot(
        x_ref[...], w_ref[...], preferred_element_type=jnp.float32, precision=_HI
    )


def _tc2_body(a0_ref, a1_ref, disnl_ref, b_ref, w_ref, p_ref):
    disnl = disnl_ref[...]
    x1 = jnp.maximum(disnl * (a0_ref[...] + a1_ref[...]) + b_ref[...], 0.0)
    p_ref[...] = disnl * jnp.dot(
        x1, w_ref[...], preferred_element_type=jnp.float32, precision=_HI
    )


def _tc3_body(a0_ref, a1_ref, disnl_ref, diswl_ref, b_ref, p_ref):
    x2 = jnp.maximum(
        disnl_ref[...] * (a0_ref[...] + a1_ref[...]) + b_ref[...], 0.0
    )
    p_ref[...] = diswl_ref[...] * x2


def _tc4_body(a0_ref, a1_ref, p2_ref, diswl_ref, w_ref, b_ref, o_ref):
    q = diswl_ref[...] * (a0_ref[...] + a1_ref[...] + p2_ref[...])
    o_ref[...] = (
        jnp.dot(q, w_ref[...], preferred_element_type=jnp.float32, precision=_HI)
        + b_ref[...]
    )


def _call(body, n_out, out_shapes):
    return pl.pallas_call(
        body,
        out_shape=[jax.ShapeDtypeStruct(s, jnp.float32) for s in out_shapes]
        if n_out > 1
        else jax.ShapeDtypeStruct(out_shapes[0], jnp.float32),
    )


def kernel(x, edge_index, W0, b0, W1, b1, W2, b2):
    srcr = edge_index[0].reshape(_TILES, _NG, _W, _ACH)
    dstr = edge_index[1].reshape(_TILES, _NG, _W, _ACH)
    zvec = jnp.zeros((N,), jnp.float32)
    zmat = jnp.zeros((N, D), jnp.float32)

    degp = _sc_deg()(dstr, zvec)
    dega = degp[0][:, None]
    degb = degp[1][:, None]

    p0, disnl, diswl = _call(_tc1_body, 3, [(N, D), (N, 1), (N, 1)])(
        x, W0, dega, degb
    )

    a1 = _sc_agg()(p0, srcr, dstr, zmat)
    p1 = _call(_tc2_body, 1, [(N, D)])(a1[0], a1[1], disnl, b0[None, :], W1)

    a2 = _sc_agg()(p1, srcr, dstr, zmat)
    p2 = _call(_tc3_body, 1, [(N, D)])(a2[0], a2[1], disnl, diswl, b1[None, :])

    a3 = _sc_agg()(p2, srcr, dstr, zmat)
    out = _call(_tc4_body, 1, [(N, NCLS)])(
        a3[0], a3[1], p2, diswl, W2, b2[None, :]
    )
    return out
